# Initial kernel scaffold; baseline (speedup 1.0000x reference)
import jax
import jax.numpy as jnp
from jax.experimental import pallas as pl


def kernel(nodes, children, feature_size):
    raise NotImplementedError("write your pallas kernel here")


# speedup vs baseline: 33.9431x; 33.9431x over previous
import jax
import jax.numpy as jnp
from jax import lax
from jax.experimental import pallas as pl
from jax.experimental.pallas import tpu as pltpu
from jax.experimental.pallas import tpu_sc as plsc

B, N, C, F = 16, 2048, 8, 128
R = B * N * C
NW = 32
RPW = R // NW
CHUNK = 128
NCHUNK = RPW // CHUNK
NB = 4


def _body(table_hbm, ch2_hbm, out_hbm, idx2, bufs, gsem, wsem):
    nc = 2
    wid = lax.axis_index("s") * nc + lax.axis_index("c")
    base = (wid // 2) * N
    row0 = wid * RPW
    chrow0 = wid * NCHUNK

    pltpu.sync_copy(ch2_hbm.at[pl.ds(chrow0, NCHUNK)], idx2)

    def fix(j, carry):
        for v in range(CHUNK // 16):
            idx2[j, pl.ds(v * 16, 16)] = idx2[j, pl.ds(v * 16, 16)] + base
        return carry

    lax.fori_loop(0, NCHUNK, fix, 0)

    zrow = jnp.zeros((16,), jnp.float32)

    def g_copy(g, b):
        return pltpu.make_async_copy(
            table_hbm.at[idx2.at[g]], bufs.at[b], gsem.at[b])

    def w_copy(g, b):
        return pltpu.make_async_copy(
            bufs.at[b], out_hbm.at[pl.ds(row0 + g * CHUNK, CHUNK)], wsem.at[b])

    def fixup(g, b):
        zc = idx2[g, pl.ds(0, 16)]
        for v in range(1, CHUNK // 16):
            zc = jnp.minimum(zc, idx2[g, pl.ds(v * 16, 16)])
        m = zc[0]
        for l in range(1, 16):
            m = jnp.minimum(m, zc[l])

        @pl.when(m == base)
        def _fix():
            def per_vreg(v, carry):
                iv = idx2[g, pl.ds(v * 16, 16)]
                for r in range(16):
                    @pl.when(iv[r] == base)
                    def _zero_row(r=r):
                        row = v * 16 + r
                        for w in range(F // 16):
                            bufs[b, row, pl.ds(w * 16, 16)] = zrow
                return carry

            lax.fori_loop(0, CHUNK // 16, per_vreg, 0)

    for b in range(NB):
        g_copy(b, b).start()

    def step(it, carry):
        for b in range(NB):
            g = it * NB + b
            g_copy(g, b).wait()
            fixup(g, b)
            w_copy(g, b).start()
            nxt = g + NB

            @pl.when(nxt < NCHUNK)
            def _refill():
                w_copy(g, b).wait()
                g_copy(nxt, b).start()
        return carry

    lax.fori_loop(0, NCHUNK // NB, step, 0)

    for b in range(NB):
        w_copy(NCHUNK - NB + b, b).wait()


def kernel(nodes, children, feature_size):
    table = nodes.reshape(B * N, F)
    ch2 = children.astype(jnp.int32).reshape(NW * NCHUNK, CHUNK)

    mesh = plsc.VectorSubcoreMesh(core_axis_name="c", subcore_axis_name="s")
    run = pl.kernel(
        _body,
        out_type=jax.ShapeDtypeStruct((R, F), jnp.float32),
        mesh=mesh,
        scratch_types=[
            pltpu.VMEM((NCHUNK, CHUNK), jnp.int32),
            pltpu.VMEM((NB, CHUNK, F), jnp.float32),
            pltpu.SemaphoreType.DMA((NB,)),
            pltpu.SemaphoreType.DMA((NB,)),
        ],
    )
    out = run(table, ch2)
    return out.reshape(B, N, C, F)
